# Initial kernel scaffold; baseline (speedup 1.0000x reference)
import jax
import jax.numpy as jnp
from jax.experimental import pallas as pl


def kernel(node_feature, edge_index, edge_types, eigvec, atom_W, atom_b, edge_table, lap_W, order_table, graph_token, null_token):
    raise NotImplementedError("write your pallas kernel here")


# speedup vs baseline: 34.0025x; 34.0025x over previous
import functools

import jax
import jax.numpy as jnp
from jax import lax
from jax.experimental import pallas as pl
from jax.experimental.pallas import tpu as pltpu
from jax.experimental.pallas import tpu_sc as plsc

B = 8
N_PER = 1024
E_PER = 4096
T2 = N_PER + E_PER + 2
D_IN = 512
HIDDEN = 1024
LAP_K = 16
NUM_EDGE_TYPES = 7

NP = N_PER + 8
EP = E_PER + 8
BH = 128
NH = HIDDEN // BH

_SC_CORES = 2
_SC_SUBCORES = 16
_NW = _SC_CORES * _SC_SUBCORES
_CHUNK = (B * E_PER) // _NW
_WPG = E_PER // _CHUNK


def _sc_gather_body(gsrc_hbm, gdst_hbm, eig_hbm, es_out, ed_out,
                    idx_v, rows_v, sem):
    wid = lax.axis_index("s") * _SC_CORES + lax.axis_index("c")
    b = wid // _WPG
    q = wid % _WPG
    in_base = wid * _CHUNK
    out_base = b * EP + 2 + q * _CHUNK
    for idx_hbm, out_hbm in ((gsrc_hbm, es_out), (gdst_hbm, ed_out)):
        pltpu.sync_copy(idx_hbm.at[pl.ds(in_base, _CHUNK)], idx_v)
        pltpu.async_copy(eig_hbm.at[idx_v], rows_v, sem).wait()
        pltpu.sync_copy(rows_v, out_hbm.at[pl.ds(out_base, _CHUNK)])


def _sc_gather(gsrc, gdst, eigvec):
    mesh = plsc.VectorSubcoreMesh(core_axis_name="c", subcore_axis_name="s")
    row_ty = jax.ShapeDtypeStruct((B * EP, LAP_K), jnp.float32)
    fn = pl.kernel(
        _sc_gather_body,
        out_type=(row_ty, row_ty),
        mesh=mesh,
        compiler_params=pltpu.CompilerParams(use_tc_tiling_on_sc=False),
        scratch_types=[
            pltpu.VMEM((_CHUNK,), jnp.int32),
            pltpu.VMEM((_CHUNK, LAP_K), jnp.float32),
            pltpu.SemaphoreType.DMA,
        ],
    )
    return fn(gsrc, gdst, eigvec)


def _tc_body(nf_ref, eign_ref, es_ref, ed_ref, aux_ref, atomw_ref,
             lapsum_ref, lap0_ref, lap1_ref, waux_ref, consts_ref, out_ref):
    f32 = jnp.float32
    node = (
        jnp.dot(nf_ref[0], atomw_ref[...], preferred_element_type=f32)
        + jnp.dot(eign_ref[0], lapsum_ref[...], preferred_element_type=f32)
        + consts_ref[0, :][None, :]
    )
    edge = (
        jnp.dot(es_ref[0], lap0_ref[...], preferred_element_type=f32)
        + jnp.dot(ed_ref[0], lap1_ref[...], preferred_element_type=f32)
        + jnp.dot(aux_ref[0], waux_ref[...], preferred_element_type=f32)
        + consts_ref[1, :][None, :]
    )
    out_ref[0, N_PER:T2, :] = edge[0 : T2 - N_PER]
    out_ref[0, 0 : N_PER + 2, :] = node[0 : N_PER + 2]
    out_ref[0, 0:2, :] = consts_ref[2:4, :]


def kernel(node_feature, edge_index, edge_types, eigvec, atom_W, atom_b,
           edge_table, lap_W, order_table, graph_token, null_token):
    f32 = jnp.float32

    src = edge_index[0].astype(jnp.int32)
    dst = edge_index[1].astype(jnp.int32)
    goffs = (jnp.arange(B, dtype=jnp.int32) * N_PER).repeat(E_PER)
    gsrc = src + goffs
    gdst = dst + goffs

    lanes = jnp.arange(16, dtype=jnp.int32)
    onehot = (edge_types[:, None] == lanes[None, :]).astype(f32)
    order = (src == dst).astype(f32)
    aux = onehot + order[:, None] * (lanes == 7).astype(f32)[None, :]
    aux = jnp.pad(aux.reshape(B, E_PER, 16), ((0, 0), (2, 6), (0, 0)))

    nf_pad = jnp.pad(node_feature.reshape(B, N_PER, D_IN),
                     ((0, 0), (2, 6), (0, 0)))
    eign_pad = jnp.pad(eigvec.reshape(B, N_PER, LAP_K),
                       ((0, 0), (2, 6), (0, 0)))

    lap0 = lap_W[:LAP_K]
    lap1 = lap_W[LAP_K:]
    lapsum = lap0 + lap1
    waux = jnp.zeros((16, HIDDEN), f32)
    waux = waux.at[0:NUM_EDGE_TYPES].set(edge_table)
    waux = waux.at[7].set(order_table[1] - order_table[0])
    consts = jnp.zeros((8, HIDDEN), f32)
    consts = consts.at[0].set(atom_b + order_table[1])
    consts = consts.at[1].set(order_table[0])
    consts = consts.at[2].set(graph_token[0])
    consts = consts.at[3].set(null_token[0])

    es, ed = _sc_gather(gsrc, gdst, eigvec)
    es = es.reshape(B, EP, LAP_K)
    ed = ed.reshape(B, EP, LAP_K)

    grid = (B, NH)
    out = pl.pallas_call(
        _tc_body,
        grid=grid,
        in_specs=[
            pl.BlockSpec((1, NP, D_IN), lambda b, h: (b, 0, 0)),
            pl.BlockSpec((1, NP, LAP_K), lambda b, h: (b, 0, 0)),
            pl.BlockSpec((1, EP, LAP_K), lambda b, h: (b, 0, 0)),
            pl.BlockSpec((1, EP, LAP_K), lambda b, h: (b, 0, 0)),
            pl.BlockSpec((1, EP, 16), lambda b, h: (b, 0, 0)),
            pl.BlockSpec((D_IN, BH), lambda b, h: (0, h)),
            pl.BlockSpec((LAP_K, BH), lambda b, h: (0, h)),
            pl.BlockSpec((LAP_K, BH), lambda b, h: (0, h)),
            pl.BlockSpec((LAP_K, BH), lambda b, h: (0, h)),
            pl.BlockSpec((16, BH), lambda b, h: (0, h)),
            pl.BlockSpec((8, BH), lambda b, h: (0, h)),
        ],
        out_specs=pl.BlockSpec((1, T2, BH), lambda b, h: (b, 0, h)),
        out_shape=jax.ShapeDtypeStruct((B, T2, HIDDEN), f32),
        compiler_params=pltpu.CompilerParams(
            dimension_semantics=("arbitrary", "arbitrary"),
        ),
    )(nf_pad, eign_pad, es, ed, aux, atom_W, lapsum, lap0, lap1, waux,
      consts)
    return out
